# baseline (speedup 1.0000x reference)
import functools
import jax
import jax.numpy as jnp
from jax import lax
from jax.experimental import pallas as pl
from jax.experimental.pallas import tpu as pltpu
from jax.experimental.pallas import tpu_sc as plsc

N_FEAT = 1000000
N_DIM = 32
BATCH = 16384

_info = plsc.get_sparse_core_info()
_NC = _info.num_cores
_NS = _info.num_subcores
_NW = _NC * _NS
_BPW = BATCH // _NW

_mesh = plsc.VectorSubcoreMesh(core_axis_name="c", subcore_axis_name="s")


@functools.partial(
    pl.kernel,
    mesh=_mesh,
    out_type=(
        jax.ShapeDtypeStruct((BATCH,), jnp.float32),
        jax.ShapeDtypeStruct((4, BATCH // 128, 8, 128), jnp.float32),
    ),
    scratch_types=[
        pltpu.VMEM((_BPW,), jnp.int32),
        pltpu.VMEM((_BPW,), jnp.float32),
        pltpu.VMEM((_BPW, N_DIM), jnp.float32),
        pltpu.VMEM((N_DIM, 4, 128), jnp.float32),
        pltpu.SemaphoreType.DMA,
        pltpu.SemaphoreType.DMA,
        pltpu.SemaphoreType.DMA,
    ],
    compiler_params=pltpu.CompilerParams(
        use_tc_tiling_on_sc=False, needs_layout_passes=False),
)
def _lookup(idx_hbm, vect_hbm, biasf_hbm, bias_out, out4,
            idx_v, bias_v, rows_v, cols3_v, sem_v, sem_b, sem_o):
    wid = lax.axis_index("s") * _NC + lax.axis_index("c")
    base = wid * _BPW
    pltpu.sync_copy(idx_hbm.at[pl.ds(base, _BPW)], idx_v)
    cb = pltpu.async_copy(biasf_hbm.at[idx_v], bias_v, sem_b)
    cv = pltpu.async_copy(vect_hbm.at[idx_v], rows_v, sem_v)
    cv.wait()
    lanes = lax.iota(jnp.int32, 16)
    for jb in range(_BPW // 16):
        rowids = jb * 16 + lanes

        def dbody(d):
            colids = jnp.zeros((16,), jnp.int32) + d
            vals = plsc.load_gather(rows_v, [rowids, colids])
            cols3_v[d, jb // 8, pl.ds((jb % 8) * 16, 16)] = vals

        pl.loop(0, N_DIM)(dbody)
    copies = []
    for d in range(N_DIM):
        g, r = d // 8, d % 8
        copies.append(pltpu.async_copy(
            cols3_v.at[d], out4.at[g, pl.ds(4 * wid, 4), r], sem_o))
    for c in copies:
        c.wait()
    cb.wait()
    pltpu.sync_copy(bias_v, bias_out.at[pl.ds(base, _BPW)])


def kernel(index, vect, bias):
    idx = index.astype(jnp.int32)
    bias_out, out4 = _lookup(idx, vect, bias[:, 0])
    return (bias_out.reshape(BATCH, 1),
            out4.transpose(1, 3, 0, 2).reshape(BATCH, N_DIM))
